# baseline (speedup 1.0000x reference)
import jax
import jax.numpy as jnp
from jax import lax
from jax.experimental import pallas as pl
from jax.experimental.pallas import tpu as pltpu
from jax.experimental.pallas import tpu_sc as plsc

NUM_CORES = 1
NUM_SUBCORES = 16
LANES = 16
NW = NUM_CORES * NUM_SUBCORES

BATCH = 16384
EMBED_DIM = 64
B_PER_W = BATCH // NW
PASS_ROWS = 512
NPASS = B_PER_W // PASS_ROWS
CHUNK = 128
NCHUNK = PASS_ROWS // CHUNK
DCH = EMBED_DIM // LANES


def _mf_kernel(user_hbm, item_hbm, utab_hbm, itab_hbm, w_hbm, b_hbm, out_hbm,
               uidx_v, iidx_v, urows_v, irows_v, w_v, b_v, out_v, sem):
  wid = lax.axis_index("s") * NUM_CORES + lax.axis_index("c")

  pltpu.sync_copy(w_hbm, w_v)
  pltpu.sync_copy(b_hbm, b_v)
  wvecs = [w_v[pl.ds(c * LANES, LANES)] for c in range(DCH)]
  bias_vec = b_v[...]
  lane = lax.iota(jnp.int32, LANES)

  for p in range(NPASS):
    pltpu.sync_copy(user_hbm.at[wid, p], uidx_v)
    pltpu.sync_copy(item_hbm.at[wid, p], iidx_v)

    copies = []
    for j in range(NCHUNK):
      copies.append(pltpu.async_copy(
          utab_hbm.at[uidx_v.at[j]], urows_v.at[pl.ds(j * CHUNK, CHUNK)], sem))
      copies.append(pltpu.async_copy(
          itab_hbm.at[iidx_v.at[j]], irows_v.at[pl.ds(j * CHUNK, CHUNK)], sem))
    for c in copies:
      c.wait()

    def body(g, _):
      row_idx = g * LANES + lane
      acc = jnp.zeros((LANES,), jnp.float32)
      for d in range(EMBED_DIM):
        col = jnp.full((LANES,), d, jnp.int32)
        u = plsc.load_gather(urows_v, [row_idx, col])
        iv = plsc.load_gather(irows_v, [row_idx, col])
        acc = acc + (u * iv) * wvecs[d // LANES][d % LANES]
      out_v[pl.ds(g * LANES, LANES)] = acc + bias_vec
      return 0

    lax.fori_loop(0, PASS_ROWS // LANES, body, 0)
    pltpu.sync_copy(out_v, out_hbm.at[wid, p])


def kernel(user, item, user_table, item_table, fc_w, fc_b):
  user4 = user.astype(jnp.int32).reshape(NW, NPASS, NCHUNK, CHUNK)
  item4 = item.astype(jnp.int32).reshape(NW, NPASS, NCHUNK, CHUNK)
  w = fc_w.reshape(EMBED_DIM)
  b = jnp.broadcast_to(fc_b.reshape(1), (LANES,))

  mesh = plsc.VectorSubcoreMesh(core_axis_name="c", subcore_axis_name="s",
                                num_cores=NUM_CORES, num_subcores=NUM_SUBCORES)
  run = pl.kernel(
      _mf_kernel,
      out_type=jax.ShapeDtypeStruct((NW, NPASS, PASS_ROWS), jnp.float32),
      mesh=mesh,
      compiler_params=pltpu.CompilerParams(needs_layout_passes=False,
                                           use_tc_tiling_on_sc=False),
      scratch_types=[
          pltpu.VMEM((NCHUNK, CHUNK), jnp.int32),
          pltpu.VMEM((NCHUNK, CHUNK), jnp.int32),
          pltpu.VMEM((PASS_ROWS, EMBED_DIM), jnp.float32),
          pltpu.VMEM((PASS_ROWS, EMBED_DIM), jnp.float32),
          pltpu.VMEM((EMBED_DIM,), jnp.float32),
          pltpu.VMEM((LANES,), jnp.float32),
          pltpu.VMEM((PASS_ROWS,), jnp.float32),
          pltpu.SemaphoreType.DMA,
      ],
  )

  out = run(user4, item4, user_table, item_table, w, b)
  return out.reshape(BATCH)


# speedup vs baseline: 1.0245x; 1.0245x over previous
import jax
import jax.numpy as jnp
from jax import lax
from jax.experimental import pallas as pl
from jax.experimental.pallas import tpu as pltpu
from jax.experimental.pallas import tpu_sc as plsc

NUM_CORES = 2
NUM_SUBCORES = 16
LANES = 16
NW = NUM_CORES * NUM_SUBCORES

BATCH = 16384
EMBED_DIM = 64
PACKED_W = 2 * EMBED_DIM
B_PER_W = BATCH // NW
PASS_ROWS = 256
NPASS = B_PER_W // PASS_ROWS
CHUNK = 128
NCHUNK = PASS_ROWS // CHUNK
DCH = EMBED_DIM // LANES


def _mf_kernel(ujdx_hbm, ijdx_hbm, uodd_hbm, iodd_hbm, utab_hbm, itab_hbm,
               w_hbm, b_hbm, out_hbm,
               ujdx_v, ijdx_v, uodd_v, iodd_v, urows_v, irows_v,
               w_v, b_v, out_v, sem):
  wid = lax.axis_index("s") * NUM_CORES + lax.axis_index("c")

  pltpu.sync_copy(w_hbm, w_v)
  pltpu.sync_copy(b_hbm, b_v)
  wvecs = [w_v[pl.ds(c * LANES, LANES)] for c in range(DCH)]
  bias_vec = b_v[...]
  lane = lax.iota(jnp.int32, LANES)

  for p in range(NPASS):
    pltpu.sync_copy(ujdx_hbm.at[wid, p], ujdx_v)
    pltpu.sync_copy(ijdx_hbm.at[wid, p], ijdx_v)
    pltpu.sync_copy(uodd_hbm.at[wid, p], uodd_v)
    pltpu.sync_copy(iodd_hbm.at[wid, p], iodd_v)

    copies = []
    for j in range(NCHUNK):
      copies.append(pltpu.async_copy(
          utab_hbm.at[ujdx_v.at[j]], urows_v.at[pl.ds(j * CHUNK, CHUNK)], sem))
      copies.append(pltpu.async_copy(
          itab_hbm.at[ijdx_v.at[j]], irows_v.at[pl.ds(j * CHUNK, CHUNK)], sem))
    for c in copies:
      c.wait()

    def body(g, _):
      row_idx = g * LANES + lane
      ucol0 = plsc.load_gather(uodd_v, [row_idx]) * EMBED_DIM
      icol0 = plsc.load_gather(iodd_v, [row_idx]) * EMBED_DIM
      acc = jnp.zeros((LANES,), jnp.float32)
      for d in range(EMBED_DIM):
        u = plsc.load_gather(urows_v, [row_idx, ucol0 + d])
        iv = plsc.load_gather(irows_v, [row_idx, icol0 + d])
        acc = acc + (u * iv) * wvecs[d // LANES][d % LANES]
      out_v[pl.ds(g * LANES, LANES)] = acc + bias_vec
      return 0

    lax.fori_loop(0, PASS_ROWS // LANES, body, 0)
    pltpu.sync_copy(out_v, out_hbm.at[wid, p])


def kernel(user, item, user_table, item_table, fc_w, fc_b):
  user = user.astype(jnp.int32)
  item = item.astype(jnp.int32)
  ujdx = (user >> 1).reshape(NW, NPASS, NCHUNK, CHUNK)
  ijdx = (item >> 1).reshape(NW, NPASS, NCHUNK, CHUNK)
  uodd = (user & 1).reshape(NW, NPASS, PASS_ROWS)
  iodd = (item & 1).reshape(NW, NPASS, PASS_ROWS)
  utab = user_table.reshape(-1, PACKED_W)
  itab = item_table.reshape(-1, PACKED_W)
  w = fc_w.reshape(EMBED_DIM)
  b = jnp.broadcast_to(fc_b.reshape(1), (LANES,))

  mesh = plsc.VectorSubcoreMesh(core_axis_name="c", subcore_axis_name="s",
                                num_cores=NUM_CORES, num_subcores=NUM_SUBCORES)
  run = pl.kernel(
      _mf_kernel,
      out_type=jax.ShapeDtypeStruct((NW, NPASS, PASS_ROWS), jnp.float32),
      mesh=mesh,
      compiler_params=pltpu.CompilerParams(needs_layout_passes=False,
                                           use_tc_tiling_on_sc=True),
      scratch_types=[
          pltpu.VMEM((NCHUNK, CHUNK), jnp.int32),
          pltpu.VMEM((NCHUNK, CHUNK), jnp.int32),
          pltpu.VMEM((PASS_ROWS,), jnp.int32),
          pltpu.VMEM((PASS_ROWS,), jnp.int32),
          pltpu.VMEM((PASS_ROWS, PACKED_W), jnp.float32),
          pltpu.VMEM((PASS_ROWS, PACKED_W), jnp.float32),
          pltpu.VMEM((EMBED_DIM,), jnp.float32),
          pltpu.VMEM((LANES,), jnp.float32),
          pltpu.VMEM((PASS_ROWS,), jnp.float32),
          pltpu.SemaphoreType.DMA,
      ],
  )

  out = run(ujdx, ijdx, uodd, iodd, utab, itab, w, b)
  return out.reshape(BATCH)


# speedup vs baseline: 2.3807x; 2.3237x over previous
import jax
import jax.numpy as jnp
from jax import lax
from jax.experimental import pallas as pl
from jax.experimental.pallas import tpu as pltpu
from jax.experimental.pallas import tpu_sc as plsc

NUM_CORES = 2
NUM_SUBCORES = 16
LANES = 16
NW = NUM_CORES * NUM_SUBCORES

BATCH = 16384
EMBED_DIM = 64
NROWS = 1000000
B_PER_W = BATCH // NW
SLAB_W = 128
NSLAB = (NROWS + SLAB_W - 1) // SLAB_W
LAST_SLAB = NSLAB - 1
RING = 8
DCH = EMBED_DIM // LANES
SEG = 256
NSEG = BATCH // SEG
SLOTS = SEG
EPAD = 264


def _gather_phase(tab_hbm, jlist_v, estart_v, col_v, nf_v, rows_v, ring_v, sem,
                  wvecs):
  zeros16 = jnp.zeros((LANES,), jnp.int32)
  nfetch = plsc.load_gather(nf_v, [zeros16])[0]

  def fire(n):
    j = plsc.load_gather(jlist_v, [jnp.full((LANES,), n, jnp.int32)])[0]
    off = pl.multiple_of(j * SLAB_W, SLAB_W)
    pltpu.async_copy(tab_hbm.at[:, pl.ds(off, SLAB_W)], ring_v.at[n % RING],
                     sem)

  def wait_for(n):
    pltpu.make_async_copy(tab_hbm.at[:, pl.ds(0, SLAB_W)],
                          ring_v.at[n % RING], sem).wait()

  for n in range(RING):
    @pl.when(n < nfetch)
    def _(n=n):
      fire(n)

  def fetch_body(n, _):
    wait_for(n)
    slot = n % RING
    slotv = jnp.full((LANES,), slot, jnp.int32)
    e_lo = plsc.load_gather(estart_v, [jnp.full((LANES,), n, jnp.int32)])[0]
    e_hi = plsc.load_gather(estart_v, [jnp.full((LANES,), n + 1, jnp.int32)])[0]

    def elem_body(e, _):
      c = plsc.load_gather(col_v, [jnp.full((LANES,), e, jnp.int32)])[0]
      cv = jnp.full((LANES,), c, jnp.int32)
      for dc in range(DCH):
        row = lax.iota(jnp.int32, LANES) + dc * LANES
        v = plsc.load_gather(ring_v, [slotv, row, cv])
        if wvecs is not None:
          v = v * wvecs[dc]
        rows_v[e, pl.ds(dc * LANES, LANES)] = v
      return 0

    lax.fori_loop(e_lo, e_hi, elem_body, 0)

    @pl.when(n + RING < nfetch)
    def _():
      fire(n + RING)
    return 0

  lax.fori_loop(0, nfetch, fetch_body, 0)


def _k1_weighted(jl_hbm, js_hbm, jc_hbm, jn_hbm, tabT_hbm, w_hbm, g_hbm,
                 jlist_v, estart_v, col_v, nf_v, rows_v, ring_v, w_v, sem):
  wid = lax.axis_index("s") * NUM_CORES + lax.axis_index("c")
  pltpu.sync_copy(w_hbm, w_v)
  wvecs = [w_v[pl.ds(c * LANES, LANES)] for c in range(DCH)]
  pltpu.sync_copy(jl_hbm.at[wid], jlist_v)
  pltpu.sync_copy(js_hbm.at[wid], estart_v)
  pltpu.sync_copy(jc_hbm.at[wid], col_v)
  pltpu.sync_copy(jn_hbm.at[wid], nf_v)
  _gather_phase(tabT_hbm, jlist_v, estart_v, col_v, nf_v, rows_v, ring_v,
                sem, wvecs)
  pltpu.sync_copy(rows_v, g_hbm.at[wid])


def _schedule(idx):
  iota = lax.iota(jnp.int32, BATCH)
  srt, order = lax.sort((idx, iota), num_keys=1)
  j = srt >> 7
  c = srt & 127
  first = (iota % SEG) == 0
  newslab = jnp.concatenate([jnp.ones((1,), jnp.bool_),
                             j[1:] != j[:-1]]) | first
  slot = jnp.cumsum(newslab.astype(jnp.int32)
                    .reshape(NSEG, SEG), axis=1) - 1
  nfetch = slot[:, -1] + 1
  srow = (iota // SEG).reshape(NSEG, SEG)
  e_local = (iota % SEG).reshape(NSEG, SEG)
  slot_or_oob = jnp.where(newslab.reshape(NSEG, SEG), slot, EPAD + 1)
  jlist = jnp.zeros((NSEG, EPAD + 8), jnp.int32).at[
      srow, slot_or_oob].set(j.reshape(NSEG, SEG), mode="drop")[:, :SLOTS]
  estart = jnp.full((NSEG, EPAD + 8), SEG, jnp.int32).at[
      srow, slot_or_oob].set(e_local, mode="drop")[:, :EPAD]
  inv = jnp.zeros((BATCH,), jnp.int32).at[order].set(iota)
  return c.reshape(NSEG, SEG), jlist, estart, nfetch, inv


def _k1_plain(jl_hbm, js_hbm, jc_hbm, jn_hbm, tabT_hbm, g_hbm,
              jlist_v, estart_v, col_v, nf_v, rows_v, ring_v, sem):
  wid = lax.axis_index("s") * NUM_CORES + lax.axis_index("c")
  pltpu.sync_copy(jl_hbm.at[wid], jlist_v)
  pltpu.sync_copy(js_hbm.at[wid], estart_v)
  pltpu.sync_copy(jc_hbm.at[wid], col_v)
  pltpu.sync_copy(jn_hbm.at[wid], nf_v)
  _gather_phase(tabT_hbm, jlist_v, estart_v, col_v, nf_v, rows_v, ring_v,
                sem, None)
  pltpu.sync_copy(rows_v, g_hbm.at[wid])


def _k2_kernel(uinv_hbm, iinv_hbm, ug_hbm, ig_hbm, b_hbm, out_hbm,
               uidx_v, iidx_v, urows_v, irows_v, b_v, out_v, sem):
  wid = lax.axis_index("s") * NUM_CORES + lax.axis_index("c")
  pltpu.sync_copy(b_hbm, b_v)
  bias_vec = b_v[...]
  lane = lax.iota(jnp.int32, LANES)

  pltpu.sync_copy(uinv_hbm.at[wid], uidx_v)
  pltpu.sync_copy(iinv_hbm.at[wid], iidx_v)

  copies = []
  for j in range(B_PER_W // 128):
    copies.append(pltpu.async_copy(
        ug_hbm.at[uidx_v.at[j]], urows_v.at[pl.ds(j * 128, 128)], sem))
    copies.append(pltpu.async_copy(
        ig_hbm.at[iidx_v.at[j]], irows_v.at[pl.ds(j * 128, 128)], sem))
  for c in copies:
    c.wait()

  def body(g, _):
    row_idx = g * LANES + lane
    acc = jnp.zeros((LANES,), jnp.float32)
    for d in range(EMBED_DIM):
      col = jnp.full((LANES,), d, jnp.int32)
      u = plsc.load_gather(urows_v, [row_idx, col])
      iv = plsc.load_gather(irows_v, [row_idx, col])
      acc = acc + u * iv
    out_v[pl.ds(g * LANES, LANES)] = acc + bias_vec
    return 0

  lax.fori_loop(0, B_PER_W // LANES, body, 0)
  pltpu.sync_copy(out_v, out_hbm.at[wid])


def kernel(user, item, user_table, item_table, fc_w, fc_b):
  user = user.astype(jnp.int32)
  item = item.astype(jnp.int32)
  ucol, ujlist, uestart, unf, uinv = _schedule(user)
  icol, ijlist, iestart, inf_, iinv = _schedule(item)
  unf8 = jnp.pad(unf.reshape(NSEG, 1), ((0, 0), (0, 7)))
  inf8 = jnp.pad(inf_.reshape(NSEG, 1), ((0, 0), (0, 7)))
  utabT = user_table.T
  itabT = item_table.T
  w = fc_w.reshape(EMBED_DIM)
  b = jnp.broadcast_to(fc_b.reshape(1), (LANES,))

  mesh = plsc.VectorSubcoreMesh(core_axis_name="c", subcore_axis_name="s",
                                num_cores=NUM_CORES, num_subcores=NUM_SUBCORES)

  k1_scratch = [
      pltpu.VMEM((SLOTS,), jnp.int32),
      pltpu.VMEM((EPAD,), jnp.int32),
      pltpu.VMEM((SEG,), jnp.int32),
      pltpu.VMEM((8,), jnp.int32),
      pltpu.VMEM((SEG, EMBED_DIM), jnp.float32),
      pltpu.VMEM((RING, EMBED_DIM, SLAB_W), jnp.float32),
  ]
  k1_params = pltpu.CompilerParams(needs_layout_passes=False,
                                   use_tc_tiling_on_sc=True)
  k1u = pl.kernel(
      _k1_weighted,
      out_type=jax.ShapeDtypeStruct((NW, SEG, EMBED_DIM), jnp.float32),
      mesh=mesh,
      compiler_params=k1_params,
      scratch_types=k1_scratch + [pltpu.VMEM((EMBED_DIM,), jnp.float32),
                                  pltpu.SemaphoreType.DMA],
  )
  k1i = pl.kernel(
      _k1_plain,
      out_type=jax.ShapeDtypeStruct((NW, SEG, EMBED_DIM), jnp.float32),
      mesh=mesh,
      compiler_params=k1_params,
      scratch_types=k1_scratch + [pltpu.SemaphoreType.DMA],
  )
  ug = jnp.stack([k1u(ujlist[0::2], uestart[0::2], ucol[0::2], unf8[0::2],
                      utabT, w),
                  k1u(ujlist[1::2], uestart[1::2], ucol[1::2], unf8[1::2],
                      utabT, w)], axis=1)
  ig = jnp.stack([k1i(ijlist[0::2], iestart[0::2], icol[0::2], inf8[0::2],
                      itabT),
                  k1i(ijlist[1::2], iestart[1::2], icol[1::2], inf8[1::2],
                      itabT)], axis=1)

  k2 = pl.kernel(
      _k2_kernel,
      out_type=jax.ShapeDtypeStruct((NW, B_PER_W), jnp.float32),
      mesh=mesh,
      compiler_params=pltpu.CompilerParams(needs_layout_passes=False,
                                           use_tc_tiling_on_sc=False),
      scratch_types=[
          pltpu.VMEM((B_PER_W // 128, 128), jnp.int32),
          pltpu.VMEM((B_PER_W // 128, 128), jnp.int32),
          pltpu.VMEM((B_PER_W, EMBED_DIM), jnp.float32),
          pltpu.VMEM((B_PER_W, EMBED_DIM), jnp.float32),
          pltpu.VMEM((LANES,), jnp.float32),
          pltpu.VMEM((B_PER_W,), jnp.float32),
          pltpu.SemaphoreType.DMA,
      ],
  )
  out = k2(uinv.reshape(NW, B_PER_W // 128, 128),
           iinv.reshape(NW, B_PER_W // 128, 128),
           ug.reshape(BATCH, EMBED_DIM), ig.reshape(BATCH, EMBED_DIM), b)
  return out.reshape(BATCH)
